# baseline (device time: 26365 ns/iter reference)
import jax
import jax.numpy as jnp
from jax import lax
from jax.experimental import pallas as pl
from jax.experimental.pallas import tpu as pltpu

N_DEV = 4
P = 2


def kernel(x, w_mat):
    m_per, k = x.shape
    _, n_per = w_mat.shape
    half = m_per // 2
    qrows = half // P

    def body(x_hbm, w_hbm, out_hbm,
             x_ref, w_ref, out_ref,
             lt_ref, lb_ref, dt_ref,
             rb_ref, rt_ref, db_ref,
             in_sems, out_sems,
             lt_s, lt_r, lb_s, lb_r, dt_s, dt_r,
             rb_s, rb_r, rt_s, rt_r, db_s, db_r):
        my_pos = lax.axis_index("i")
        left = lax.rem(my_pos + N_DEV - 1, N_DEV)
        right = lax.rem(my_pos + 1, N_DEV)

        x_in = pltpu.make_async_copy(x_hbm, x_ref, in_sems.at[0])
        w_in = pltpu.make_async_copy(w_hbm, w_ref, in_sems.at[1])
        x_in.start()
        w_in.start()

        barrier_sem = pltpu.get_barrier_semaphore()
        for nbr in (left, right):
            pl.semaphore_signal(
                barrier_sem, inc=1,
                device_id=(nbr,), device_id_type=pl.DeviceIdType.MESH,
            )
        pl.semaphore_wait(barrier_sem, 2)

        def rcopy(src, dst, ssem, rsem, dev):
            return pltpu.make_async_remote_copy(
                src_ref=src, dst_ref=dst, send_sem=ssem, recv_sem=rsem,
                device_id=(dev,), device_id_type=pl.DeviceIdType.MESH,
            )

        lt = [rcopy(x_ref.at[pl.ds(q * qrows, qrows)],
                    lt_ref.at[pl.ds(q * qrows, qrows)],
                    lt_s.at[q], lt_r.at[q], right) for q in range(P)]
        lb = rcopy(x_ref.at[pl.ds(half, half)], lb_ref,
                   lb_s.at[0], lb_r.at[0], right)
        rb = [rcopy(x_ref.at[pl.ds(half + q * qrows, qrows)],
                    rb_ref.at[pl.ds(q * qrows, qrows)],
                    rb_s.at[q], rb_r.at[q], left) for q in range(P)]
        rt = rcopy(x_ref.at[pl.ds(0, half)], rt_ref,
                   rt_s.at[0], rt_r.at[0], left)
        dt = [rcopy(lt_ref.at[pl.ds(q * qrows, qrows)],
                    dt_ref.at[pl.ds(q * qrows, qrows)],
                    dt_s.at[q], dt_r.at[q], right) for q in range(P)]
        db = [rcopy(rb_ref.at[pl.ds(q * qrows, qrows)],
                    db_ref.at[pl.ds(q * qrows, qrows)],
                    db_s.at[q], db_r.at[q], left) for q in range(P)]

        x_in.wait()
        for q in range(P):
            lt[q].start()
            rb[q].start()
        lb.start()
        rt.start()

        out_copies = []

        def gemm_store(data, origin, row_off, rows, sem_idx):
            sl = pl.ds(origin * m_per + row_off, rows)
            out_ref[sl, :] = jnp.maximum(
                jnp.dot(data, w_ref[:], preferred_element_type=jnp.float32),
                0.0,
            )
            cp = pltpu.make_async_copy(
                out_ref.at[sl], out_hbm.at[sl], out_sems.at[sem_idx]
            )
            cp.start()
            out_copies.append(cp)

        w_in.wait()
        gemm_store(x_ref[:], my_pos, 0, m_per, 0)

        for q in range(P):
            lt[q].wait_recv()
            dt[q].start()
            rb[q].wait_recv()
            db[q].start()

        gemm_store(lt_ref[:], left, 0, half, 1)
        gemm_store(rb_ref[:], right, half, half, 2)

        lb.wait_recv()
        rt.wait_recv()
        gemm_store(lb_ref[:], left, half, half, 3)
        gemm_store(rt_ref[:], right, 0, half, 4)

        diag = lax.rem(my_pos + 2, N_DEV)
        for q in range(P):
            dt[q].wait_recv()
            gemm_store(dt_ref[pl.ds(q * qrows, qrows), :],
                       diag, q * qrows, qrows, 5 + q)
            db[q].wait_recv()
            gemm_store(db_ref[pl.ds(q * qrows, qrows), :],
                       diag, half + q * qrows, qrows, 7 + q)

        for cp in out_copies:
            cp.wait()
        for q in range(P):
            lt[q].wait_send()
            rb[q].wait_send()
            dt[q].wait_send()
            db[q].wait_send()
        lb.wait_send()
        rt.wait_send()

    half_buf = pltpu.VMEM((half, k), jnp.float32)
    return pl.pallas_call(
        body,
        out_shape=jax.ShapeDtypeStruct((N_DEV * m_per, n_per), jnp.float32),
        in_specs=[
            pl.BlockSpec(memory_space=pltpu.ANY),
            pl.BlockSpec(memory_space=pltpu.ANY),
        ],
        out_specs=pl.BlockSpec(memory_space=pltpu.ANY),
        scratch_shapes=[
            pltpu.VMEM((m_per, k), jnp.float32),
            pltpu.VMEM((k, n_per), jnp.float32),
            pltpu.VMEM((N_DEV * m_per, n_per), jnp.float32),
            half_buf, half_buf, half_buf,
            half_buf, half_buf, half_buf,
            pltpu.SemaphoreType.DMA((2,)),
            pltpu.SemaphoreType.DMA((9,)),
            pltpu.SemaphoreType.DMA((P,)), pltpu.SemaphoreType.DMA((P,)),
            pltpu.SemaphoreType.DMA((1,)), pltpu.SemaphoreType.DMA((1,)),
            pltpu.SemaphoreType.DMA((P,)), pltpu.SemaphoreType.DMA((P,)),
            pltpu.SemaphoreType.DMA((P,)), pltpu.SemaphoreType.DMA((P,)),
            pltpu.SemaphoreType.DMA((1,)), pltpu.SemaphoreType.DMA((1,)),
            pltpu.SemaphoreType.DMA((P,)), pltpu.SemaphoreType.DMA((P,)),
        ],
        compiler_params=pltpu.CompilerParams(collective_id=0),
    )(x, w_mat)


# device time: 17488 ns/iter; 1.5076x vs baseline; 1.5076x over previous
import jax
import jax.numpy as jnp
from jax import lax
from jax.experimental import pallas as pl
from jax.experimental.pallas import tpu as pltpu

N_DEV = 4
P = 2


def kernel(x, w_mat):
    m_per, k = x.shape
    _, n_per = w_mat.shape
    half = m_per // 2
    qrows = half // P

    def body(x_ref, w_ref, out_ref,
             xq_ref,
             lt_ref, lb_ref, dt_ref,
             rb_ref, rt_ref, db_ref,
             lt_s, lt_r, lb_s, lb_r, dt_s, dt_r,
             rb_s, rb_r, rt_s, rt_r, db_s, db_r):
        my_pos = lax.axis_index("i")
        left = lax.rem(my_pos + N_DEV - 1, N_DEV)
        right = lax.rem(my_pos + 1, N_DEV)

        barrier_sem = pltpu.get_barrier_semaphore()
        for nbr in (left, right):
            pl.semaphore_signal(
                barrier_sem, inc=1,
                device_id=(nbr,), device_id_type=pl.DeviceIdType.MESH,
            )
        xq_ref[:] = x_ref[:].astype(jnp.bfloat16)
        pl.semaphore_wait(barrier_sem, 2)

        def rcopy(src, dst, ssem, rsem, dev):
            return pltpu.make_async_remote_copy(
                src_ref=src, dst_ref=dst, send_sem=ssem, recv_sem=rsem,
                device_id=(dev,), device_id_type=pl.DeviceIdType.MESH,
            )

        lt = [rcopy(xq_ref.at[pl.ds(q * qrows, qrows)],
                    lt_ref.at[pl.ds(q * qrows, qrows)],
                    lt_s.at[q], lt_r.at[q], right) for q in range(P)]
        lb = rcopy(xq_ref.at[pl.ds(half, half)], lb_ref,
                   lb_s.at[0], lb_r.at[0], right)
        rb = [rcopy(xq_ref.at[pl.ds(half + q * qrows, qrows)],
                    rb_ref.at[pl.ds(q * qrows, qrows)],
                    rb_s.at[q], rb_r.at[q], left) for q in range(P)]
        rt = rcopy(xq_ref.at[pl.ds(0, half)], rt_ref,
                   rt_s.at[0], rt_r.at[0], left)
        dt = [rcopy(lt_ref.at[pl.ds(q * qrows, qrows)],
                    dt_ref.at[pl.ds(q * qrows, qrows)],
                    dt_s.at[q], dt_r.at[q], right) for q in range(P)]
        db = [rcopy(rb_ref.at[pl.ds(q * qrows, qrows)],
                    db_ref.at[pl.ds(q * qrows, qrows)],
                    db_s.at[q], db_r.at[q], left) for q in range(P)]

        for q in range(P):
            lt[q].start()
            rb[q].start()
        lb.start()
        rt.start()

        def gemm_store(data, origin, row_off, rows):
            out_ref[pl.ds(origin * m_per + row_off, rows), :] = jnp.maximum(
                jnp.dot(data.astype(jnp.float32), w_ref[:],
                        preferred_element_type=jnp.float32),
                0.0,
            )

        gemm_store(x_ref[:], my_pos, 0, m_per)

        for q in range(P):
            lt[q].wait_recv()
            dt[q].start()
            rb[q].wait_recv()
            db[q].start()

        gemm_store(lt_ref[:], left, 0, half)
        gemm_store(rb_ref[:], right, half, half)

        lb.wait_recv()
        rt.wait_recv()
        gemm_store(lb_ref[:], left, half, half)
        gemm_store(rt_ref[:], right, 0, half)

        diag = lax.rem(my_pos + 2, N_DEV)
        for q in range(P):
            dt[q].wait_recv()
            gemm_store(dt_ref[pl.ds(q * qrows, qrows), :],
                       diag, q * qrows, qrows)
            db[q].wait_recv()
            gemm_store(db_ref[pl.ds(q * qrows, qrows), :],
                       diag, half + q * qrows, qrows)

        for q in range(P):
            lt[q].wait_send()
            rb[q].wait_send()
            dt[q].wait_send()
            db[q].wait_send()
        lb.wait_send()
        rt.wait_send()

    half_buf = pltpu.VMEM((half, k), jnp.bfloat16)
    return pl.pallas_call(
        body,
        out_shape=jax.ShapeDtypeStruct((N_DEV * m_per, n_per), jnp.float32),
        in_specs=[
            pl.BlockSpec(memory_space=pltpu.VMEM),
            pl.BlockSpec(memory_space=pltpu.VMEM),
        ],
        out_specs=pl.BlockSpec(memory_space=pltpu.VMEM),
        scratch_shapes=[
            pltpu.VMEM((m_per, k), jnp.bfloat16),
            half_buf, half_buf, half_buf,
            half_buf, half_buf, half_buf,
            pltpu.SemaphoreType.DMA((P,)), pltpu.SemaphoreType.DMA((P,)),
            pltpu.SemaphoreType.DMA((1,)), pltpu.SemaphoreType.DMA((1,)),
            pltpu.SemaphoreType.DMA((P,)), pltpu.SemaphoreType.DMA((P,)),
            pltpu.SemaphoreType.DMA((P,)), pltpu.SemaphoreType.DMA((P,)),
            pltpu.SemaphoreType.DMA((1,)), pltpu.SemaphoreType.DMA((1,)),
            pltpu.SemaphoreType.DMA((P,)), pltpu.SemaphoreType.DMA((P,)),
        ],
        compiler_params=pltpu.CompilerParams(collective_id=0),
    )(x, w_mat)
